# Initial kernel scaffold; baseline (speedup 1.0000x reference)
import jax
import jax.numpy as jnp
from jax import lax
from jax.experimental import pallas as pl
from jax.experimental.pallas import tpu as pltpu
from jax.experimental.pallas import tpu_sc as plsc

N = 10000
D = 128
E = 320000
NC, NS = 2, 16
NW = NC * NS
NP = 10240
EB = 128
NB = -(-E // (NW * EB))
EP = NW * NB * EB
DW = 16
RPT = NP // NS


def _agg_body(h_hbm, src_hbm, dst_hbm, s_out, deg_out,
              src_v, dst_v, rows_v, ones_v, s_acc, deg_acc, sem):
    c = lax.axis_index("c")
    s = lax.axis_index("s")
    wid = s * NC + c

    def _zrow(i, _):
        for j in range(D // 16):
            rows_v[i, pl.ds(j * 16, 16)] = jnp.zeros((16,), jnp.float32)
        ones_v[i, :] = jnp.zeros((16,), jnp.float32)
        return 0
    lax.fori_loop(0, EB, _zrow, 0)

    base = s * RPT
    for k in range(RPT // EB):
        pltpu.sync_copy(rows_v, s_acc.at[pl.ds(base + k * EB, EB)])
        pltpu.sync_copy(ones_v, deg_acc.at[pl.ds(base + k * EB, EB)])

    def _orow(i, _):
        ones_v[i, :] = jnp.ones((16,), jnp.float32)
        return 0
    lax.fori_loop(0, EB, _orow, 0)

    pltpu.sync_copy(src_hbm.at[pl.ds(wid * NB, NB)], src_v)
    pltpu.sync_copy(dst_hbm.at[pl.ds(wid * NB, NB)], dst_v)

    plsc.subcore_barrier()

    def _edge_block(j, _):
        pltpu.async_copy(h_hbm.at[src_v.at[j]], rows_v, sem).wait()
        pltpu.sync_copy(rows_v, s_acc.at[dst_v.at[j]], add=True)
        pltpu.sync_copy(ones_v, deg_acc.at[dst_v.at[j]], add=True)
        return 0
    lax.fori_loop(0, NB, _edge_block, 0)

    plsc.subcore_barrier()

    for k in range(RPT // EB):
        r0 = base + k * EB
        pltpu.sync_copy(s_acc.at[pl.ds(r0, EB)], s_out.at[c, pl.ds(r0, EB)])
        pltpu.sync_copy(deg_acc.at[pl.ds(r0, EB)], deg_out.at[c, pl.ds(r0, EB)])


_agg = pl.kernel(
    _agg_body,
    out_type=[jax.ShapeDtypeStruct((NC, NP, D), jnp.float32),
              jax.ShapeDtypeStruct((NC, NP, DW), jnp.float32)],
    mesh=plsc.VectorSubcoreMesh(core_axis_name="c", subcore_axis_name="s",
                                num_cores=NC, num_subcores=NS),
    scratch_types=[
        pltpu.VMEM((NB, EB), jnp.int32),
        pltpu.VMEM((NB, EB), jnp.int32),
        pltpu.VMEM((EB, D), jnp.float32),
        pltpu.VMEM((EB, DW), jnp.float32),
        pltpu.VMEM_SHARED((NP, D), jnp.float32),
        pltpu.VMEM_SHARED((NP, DW), jnp.float32),
        pltpu.SemaphoreType.DMA,
    ],
)

BR = 1280


def _combine_body(x_ref, s_ref, d_ref, wn_ref, ws_ref, b_ref, o_ref):
    deg = jnp.sum(d_ref[0] + d_ref[1], axis=-1, keepdims=True) * (1.0 / DW)
    invd = jnp.where(deg > 0, 1.0 / jnp.maximum(deg, 1.0), 0.0)
    mean = (s_ref[0] + s_ref[1]) * invd
    hs = lax.dot_general(x_ref[...], ws_ref[...], (((1,), (1,)), ((), ())),
                         precision=lax.Precision.HIGHEST,
                         preferred_element_type=jnp.float32)
    hn = lax.dot_general(mean, wn_ref[...], (((1,), (1,)), ((), ())),
                         precision=lax.Precision.HIGHEST,
                         preferred_element_type=jnp.float32)
    o_ref[...] = hs + hn + b_ref[...]


def _combine(h, s_parts, deg_parts, W_neigh, W_self, b):
    return pl.pallas_call(
        _combine_body,
        grid=(NP // BR,),
        in_specs=[
            pl.BlockSpec((BR, D), lambda i: (i, 0)),
            pl.BlockSpec((NC, BR, D), lambda i: (0, i, 0)),
            pl.BlockSpec((NC, BR, DW), lambda i: (0, i, 0)),
            pl.BlockSpec((D, D), lambda i: (0, 0)),
            pl.BlockSpec((D, D), lambda i: (0, 0)),
            pl.BlockSpec((1, D), lambda i: (0, 0)),
        ],
        out_specs=pl.BlockSpec((BR, D), lambda i: (i, 0)),
        out_shape=jax.ShapeDtypeStruct((NP, D), jnp.float32),
    )(h, s_parts, deg_parts, W_neigh, W_self, b.reshape(1, D))


def kernel(x, edge_index, W_neigh1, W_self1, b_self1, W_neigh2, W_self2, b_self2):
    src = edge_index[0]
    dst = edge_index[1]
    pad_ids = lax.iota(jnp.int32, EP - E)
    srcp = jnp.concatenate([src, pad_ids % NP]).reshape(EP // EB, EB)
    dstp = jnp.concatenate([dst, N + pad_ids % (NP - N)]).reshape(EP // EB, EB)
    x_pad = jnp.pad(x, ((0, NP - N), (0, 0)))

    s1, d1 = _agg(x_pad, srcp, dstp)
    h1 = _combine(x_pad, s1, d1, W_neigh1, W_self1, b_self1)
    s2, d2 = _agg(h1, srcp, dstp)
    h2 = _combine(h1, s2, d2, W_neigh2, W_self2, b_self2)
    return h2[:N]


# speedup vs baseline: 6.8509x; 6.8509x over previous
import jax
import jax.numpy as jnp
from jax import lax
from jax.experimental import pallas as pl
from jax.experimental.pallas import tpu as pltpu
from jax.experimental.pallas import tpu_sc as plsc

N = 10000
D = 128
E = 320000
NC, NS = 2, 16
NW = NC * NS
NP = 10240
EB = 128
NB = (-(-E // (NW * EB)) + 7) // 8 * 8
EP = NW * NB * EB
RPT = NP // NS
IC = 8

_MESH = plsc.VectorSubcoreMesh(core_axis_name="c", subcore_axis_name="s",
                               num_cores=NC, num_subcores=NS)


def _sum_body(h_hbm, src_hbm, dst_hbm, s_out, src_v, dst_v, rows_v, s_acc, sem):
    c = lax.axis_index("c")
    s = lax.axis_index("s")
    wid = s * NC + c

    def _zrow(i, _):
        for j in range(D // 16):
            rows_v[i, pl.ds(j * 16, 16)] = jnp.zeros((16,), jnp.float32)
        return 0
    lax.fori_loop(0, EB, _zrow, 0)

    base = s * RPT
    for k in range(RPT // EB):
        pltpu.sync_copy(rows_v, s_acc.at[pl.ds(base + k * EB, EB)])

    plsc.subcore_barrier()

    def _chunk(jc, _):
        off = wid * NB + jc * IC
        pltpu.sync_copy(src_hbm.at[pl.ds(off, IC)], src_v)
        pltpu.sync_copy(dst_hbm.at[pl.ds(off, IC)], dst_v)

        def _edge_block(j, _):
            pltpu.async_copy(h_hbm.at[src_v.at[j]], rows_v, sem).wait()
            pltpu.sync_copy(rows_v, s_acc.at[dst_v.at[j]], add=True)
            return 0
        lax.fori_loop(0, IC, _edge_block, 0)
        return 0
    lax.fori_loop(0, NB // IC, _chunk, 0)

    plsc.subcore_barrier()

    for k in range(RPT // EB):
        r0 = base + k * EB
        pltpu.sync_copy(s_acc.at[pl.ds(r0, EB)], s_out.at[pl.ds(c * NP + r0, EB)])


_sum = pl.kernel(
    _sum_body,
    out_type=jax.ShapeDtypeStruct((NC * NP, D), jnp.float32),
    mesh=_MESH,
    scratch_types=[
        pltpu.VMEM((IC, EB), jnp.int32),
        pltpu.VMEM((IC, EB), jnp.int32),
        pltpu.VMEM((EB, D), jnp.float32),
        pltpu.VMEM_SHARED((NP, D), jnp.float32),
        pltpu.SemaphoreType.DMA,
    ],
)


def _deg_body(dst_hbm, deg_out, dst_v, ones_v, deg_acc):
    c = lax.axis_index("c")
    s = lax.axis_index("s")
    wid = s * NC + c

    def _fill(val):
        def _row(i, _):
            for j in range(D // 16):
                ones_v[i, pl.ds(j * 16, 16)] = jnp.full((16,), val, jnp.float32)
            return 0
        lax.fori_loop(0, EB, _row, 0)

    _fill(0.0)
    base = s * RPT
    for k in range(RPT // EB):
        pltpu.sync_copy(ones_v, deg_acc.at[pl.ds(base + k * EB, EB)])
    _fill(1.0)

    plsc.subcore_barrier()

    def _chunk(jc, _):
        off = wid * NB + jc * IC
        pltpu.sync_copy(dst_hbm.at[pl.ds(off, IC)], dst_v)

        def _edge_block(j, _):
            pltpu.sync_copy(ones_v, deg_acc.at[dst_v.at[j]], add=True)
            return 0
        lax.fori_loop(0, IC, _edge_block, 0)
        return 0
    lax.fori_loop(0, NB // IC, _chunk, 0)

    plsc.subcore_barrier()

    for k in range(RPT // EB):
        r0 = base + k * EB
        pltpu.sync_copy(deg_acc.at[pl.ds(r0, EB)], deg_out.at[pl.ds(c * NP + r0, EB)])


_deg = pl.kernel(
    _deg_body,
    out_type=jax.ShapeDtypeStruct((NC * NP, D), jnp.float32),
    mesh=_MESH,
    scratch_types=[
        pltpu.VMEM((IC, EB), jnp.int32),
        pltpu.VMEM((EB, D), jnp.float32),
        pltpu.VMEM_SHARED((NP, D), jnp.float32),
    ],
)

BR = 1280


def _combine_body(x_ref, s_ref, d_ref, wn_ref, ws_ref, b_ref, o_ref):
    deg = d_ref[0, :, :1] + d_ref[1, :, :1]
    invd = jnp.where(deg > 0, 1.0 / jnp.maximum(deg, 1.0), 0.0)
    mean = (s_ref[0] + s_ref[1]) * invd
    hs = lax.dot_general(x_ref[...], ws_ref[...], (((1,), (1,)), ((), ())),
                         precision=lax.Precision.HIGHEST,
                         preferred_element_type=jnp.float32)
    hn = lax.dot_general(mean, wn_ref[...], (((1,), (1,)), ((), ())),
                         precision=lax.Precision.HIGHEST,
                         preferred_element_type=jnp.float32)
    o_ref[...] = hs + hn + b_ref[...]


def _combine(h, s_parts, deg_parts, W_neigh, W_self, b):
    return pl.pallas_call(
        _combine_body,
        grid=(NP // BR,),
        in_specs=[
            pl.BlockSpec((BR, D), lambda i: (i, 0)),
            pl.BlockSpec((NC, BR, D), lambda i: (0, i, 0)),
            pl.BlockSpec((NC, BR, D), lambda i: (0, i, 0)),
            pl.BlockSpec((D, D), lambda i: (0, 0)),
            pl.BlockSpec((D, D), lambda i: (0, 0)),
            pl.BlockSpec((1, D), lambda i: (0, 0)),
        ],
        out_specs=pl.BlockSpec((BR, D), lambda i: (i, 0)),
        out_shape=jax.ShapeDtypeStruct((NP, D), jnp.float32),
    )(h, s_parts.reshape(NC, NP, D), deg_parts.reshape(NC, NP, D),
      W_neigh, W_self, b.reshape(1, D))


def kernel(x, edge_index, W_neigh1, W_self1, b_self1, W_neigh2, W_self2, b_self2):
    src = edge_index[0]
    dst = edge_index[1]
    pad_ids = lax.iota(jnp.int32, EP - E)
    srcp = jnp.concatenate([src, pad_ids % NP]).reshape(EP // EB, EB)
    dstp = jnp.concatenate([dst, N + pad_ids % (NP - N)]).reshape(EP // EB, EB)
    x_pad = jnp.pad(x, ((0, NP - N), (0, 0)))

    d = _deg(dstp)
    s1 = _sum(x_pad, srcp, dstp)
    h1 = _combine(x_pad, s1, d, W_neigh1, W_self1, b_self1)
    s2 = _sum(h1, srcp, dstp)
    h2 = _combine(h1, s2, d, W_neigh2, W_self2, b_self2)
    return h2[:N]


# speedup vs baseline: 7.9145x; 1.1553x over previous
import jax
import jax.numpy as jnp
from jax import lax
from jax.experimental import pallas as pl
from jax.experimental.pallas import tpu as pltpu
from jax.experimental.pallas import tpu_sc as plsc

N = 10000
D = 128
E = 320000
NC, NS = 2, 16
NW = NC * NS
NP = 10240
EB = 64
NB = (-(-E // (NW * EB)) + 7) // 8 * 8
EP = NW * NB * EB
RPT = NP // NS
IC = 16

_MESH = plsc.VectorSubcoreMesh(core_axis_name="c", subcore_axis_name="s",
                               num_cores=NC, num_subcores=NS)


def _sum_body(h_hbm, src_hbm, dst_hbm, s_out,
              src_v, dst_v, rows_v, s_acc, g_sem0, g_sem1, s_sem0, s_sem1):
    c = lax.axis_index("c")
    s = lax.axis_index("s")
    wid = s * NC + c
    gsems = [g_sem0, g_sem1]
    ssems = [s_sem0, s_sem1]

    def _zrow(i, _):
        for j in range(D // 16):
            rows_v[0, i, pl.ds(j * 16, 16)] = jnp.zeros((16,), jnp.float32)
        return 0
    lax.fori_loop(0, EB, _zrow, 0)

    base = s * RPT
    for k in range(RPT // EB):
        pltpu.sync_copy(rows_v.at[0], s_acc.at[pl.ds(base + k * EB, EB)])

    plsc.subcore_barrier()

    def _chunk(jc, _):
        off = wid * NB + jc * IC
        pltpu.sync_copy(src_hbm.at[pl.ds(off, IC)], src_v)
        pltpu.sync_copy(dst_hbm.at[pl.ds(off, IC)], dst_v)

        g = {0: pltpu.async_copy(h_hbm.at[src_v.at[0]], rows_v.at[0], gsems[0])}
        sc = {}
        for j in range(IC):
            b = j % 2
            if j >= 1:
                sc[j - 1].wait()
            if j + 1 < IC:
                g[j + 1] = pltpu.async_copy(
                    h_hbm.at[src_v.at[j + 1]], rows_v.at[(j + 1) % 2],
                    gsems[(j + 1) % 2])
            g[j].wait()
            sc[j] = pltpu.async_copy(
                rows_v.at[b], s_acc.at[dst_v.at[j]], ssems[b], add=True)
        sc[IC - 1].wait()
        return 0
    lax.fori_loop(0, NB // IC, _chunk, 0)

    plsc.subcore_barrier()

    for k in range(RPT // EB):
        r0 = base + k * EB
        pltpu.sync_copy(s_acc.at[pl.ds(r0, EB)], s_out.at[pl.ds(c * NP + r0, EB)])


_sum = pl.kernel(
    _sum_body,
    out_type=jax.ShapeDtypeStruct((NC * NP, D), jnp.float32),
    mesh=_MESH,
    scratch_types=[
        pltpu.VMEM((IC, EB), jnp.int32),
        pltpu.VMEM((IC, EB), jnp.int32),
        pltpu.VMEM((2, EB, D), jnp.float32),
        pltpu.VMEM_SHARED((NP, D), jnp.float32),
        pltpu.SemaphoreType.DMA,
        pltpu.SemaphoreType.DMA,
        pltpu.SemaphoreType.DMA,
        pltpu.SemaphoreType.DMA,
    ],
)


def _deg_body(dst_hbm, deg_out, dst_v, ones_v, deg_acc, s_sem):
    c = lax.axis_index("c")
    s = lax.axis_index("s")
    wid = s * NC + c

    def _fill(val):
        def _row(i, _):
            for j in range(D // 16):
                ones_v[i, pl.ds(j * 16, 16)] = jnp.full((16,), val, jnp.float32)
            return 0
        lax.fori_loop(0, EB, _row, 0)

    _fill(0.0)
    base = s * RPT
    for k in range(RPT // EB):
        pltpu.sync_copy(ones_v, deg_acc.at[pl.ds(base + k * EB, EB)])
    _fill(1.0)

    plsc.subcore_barrier()

    def _chunk(jc, _):
        off = wid * NB + jc * IC
        pltpu.sync_copy(dst_hbm.at[pl.ds(off, IC)], dst_v)

        hs = [pltpu.async_copy(ones_v, deg_acc.at[dst_v.at[j]], s_sem, add=True)
              for j in range(IC)]
        for h in hs:
            h.wait()
        return 0
    lax.fori_loop(0, NB // IC, _chunk, 0)

    plsc.subcore_barrier()

    for k in range(RPT // EB):
        r0 = base + k * EB
        pltpu.sync_copy(deg_acc.at[pl.ds(r0, EB)], deg_out.at[pl.ds(c * NP + r0, EB)])


_deg = pl.kernel(
    _deg_body,
    out_type=jax.ShapeDtypeStruct((NC * NP, D), jnp.float32),
    mesh=_MESH,
    scratch_types=[
        pltpu.VMEM((IC, EB), jnp.int32),
        pltpu.VMEM((EB, D), jnp.float32),
        pltpu.VMEM_SHARED((NP, D), jnp.float32),
        pltpu.SemaphoreType.DMA,
    ],
)

BR = 1280


def _combine_body(x_ref, s_ref, d_ref, wn_ref, ws_ref, b_ref, o_ref):
    deg = d_ref[0, :, :1] + d_ref[1, :, :1]
    invd = jnp.where(deg > 0, 1.0 / jnp.maximum(deg, 1.0), 0.0)
    mean = (s_ref[0] + s_ref[1]) * invd
    hs = lax.dot_general(x_ref[...], ws_ref[...], (((1,), (1,)), ((), ())),
                         precision=lax.Precision.HIGHEST,
                         preferred_element_type=jnp.float32)
    hn = lax.dot_general(mean, wn_ref[...], (((1,), (1,)), ((), ())),
                         precision=lax.Precision.HIGHEST,
                         preferred_element_type=jnp.float32)
    o_ref[...] = hs + hn + b_ref[...]


def _combine(h, s_parts, deg_parts, W_neigh, W_self, b):
    return pl.pallas_call(
        _combine_body,
        grid=(NP // BR,),
        in_specs=[
            pl.BlockSpec((BR, D), lambda i: (i, 0)),
            pl.BlockSpec((NC, BR, D), lambda i: (0, i, 0)),
            pl.BlockSpec((NC, BR, D), lambda i: (0, i, 0)),
            pl.BlockSpec((D, D), lambda i: (0, 0)),
            pl.BlockSpec((D, D), lambda i: (0, 0)),
            pl.BlockSpec((1, D), lambda i: (0, 0)),
        ],
        out_specs=pl.BlockSpec((BR, D), lambda i: (i, 0)),
        out_shape=jax.ShapeDtypeStruct((NP, D), jnp.float32),
    )(h, s_parts.reshape(NC, NP, D), deg_parts.reshape(NC, NP, D),
      W_neigh, W_self, b.reshape(1, D))


def kernel(x, edge_index, W_neigh1, W_self1, b_self1, W_neigh2, W_self2, b_self2):
    src = edge_index[0]
    dst = edge_index[1]
    pad_ids = lax.iota(jnp.int32, EP - E)
    srcp = jnp.concatenate([src, pad_ids % NP]).reshape(EP // EB, EB)
    dstp = jnp.concatenate([dst, N + pad_ids % (NP - N)]).reshape(EP // EB, EB)
    x_pad = jnp.pad(x, ((0, NP - N), (0, 0)))

    d = _deg(dstp)
    s1 = _sum(x_pad, srcp, dstp)
    h1 = _combine(x_pad, s1, d, W_neigh1, W_self1, b_self1)
    s2 = _sum(h1, srcp, dstp)
    h2 = _combine(h1, s2, d, W_neigh2, W_self2, b_self2)
    return h2[:N]


# speedup vs baseline: 8.1319x; 1.0275x over previous
import jax
import jax.numpy as jnp
from jax import lax
from jax.experimental import pallas as pl
from jax.experimental.pallas import tpu as pltpu
from jax.experimental.pallas import tpu_sc as plsc

N = 10000
D = 128
E = 320000
NC, NS = 2, 16
NW = NC * NS
NP = 10240
EB = 64
NB = (-(-E // (NW * EB)) + 7) // 8 * 8
EP = NW * NB * EB
RPT = NP // NS
IC = 16

_MESH = plsc.VectorSubcoreMesh(core_axis_name="c", subcore_axis_name="s",
                               num_cores=NC, num_subcores=NS)


def _sum_body(h_hbm, src_hbm, dst_hbm, s_out,
              src_v, dst_v, rows_v, s_acc, g_sem0, g_sem1, s_sem0, s_sem1):
    c = lax.axis_index("c")
    s = lax.axis_index("s")
    wid = s * NC + c
    gsems = [g_sem0, g_sem1]
    ssems = [s_sem0, s_sem1]

    def _zrow(i, _):
        for j in range(D // 16):
            rows_v[0, i, pl.ds(j * 16, 16)] = jnp.zeros((16,), jnp.float32)
        return 0
    lax.fori_loop(0, EB, _zrow, 0)

    base = s * RPT
    for k in range(RPT // EB):
        pltpu.sync_copy(rows_v.at[0], s_acc.at[pl.ds(base + k * EB, EB)])

    plsc.subcore_barrier()

    def _chunk(jc, _):
        off = wid * NB + jc * IC
        pltpu.sync_copy(src_hbm.at[pl.ds(off, IC)], src_v)
        pltpu.sync_copy(dst_hbm.at[pl.ds(off, IC)], dst_v)

        g = {0: pltpu.async_copy(h_hbm.at[src_v.at[0]], rows_v.at[0], gsems[0])}
        sc = {}
        for j in range(IC):
            b = j % 2
            if j >= 1:
                sc[j - 1].wait()
            if j + 1 < IC:
                g[j + 1] = pltpu.async_copy(
                    h_hbm.at[src_v.at[j + 1]], rows_v.at[(j + 1) % 2],
                    gsems[(j + 1) % 2])
            g[j].wait()
            sc[j] = pltpu.async_copy(
                rows_v.at[b], s_acc.at[dst_v.at[j]], ssems[b], add=True)
        sc[IC - 1].wait()
        return 0
    lax.fori_loop(0, NB // IC, _chunk, 0)

    plsc.subcore_barrier()

    for k in range(RPT // EB):
        r0 = base + k * EB
        pltpu.sync_copy(s_acc.at[pl.ds(r0, EB)], s_out.at[pl.ds(c * NP + r0, EB)])


_sum = pl.kernel(
    _sum_body,
    out_type=jax.ShapeDtypeStruct((NC * NP, D), jnp.float32),
    mesh=_MESH,
    scratch_types=[
        pltpu.VMEM((IC, EB), jnp.int32),
        pltpu.VMEM((IC, EB), jnp.int32),
        pltpu.VMEM((2, EB, D), jnp.float32),
        pltpu.VMEM_SHARED((NP, D), jnp.float32),
        pltpu.SemaphoreType.DMA,
        pltpu.SemaphoreType.DMA,
        pltpu.SemaphoreType.DMA,
        pltpu.SemaphoreType.DMA,
    ],
)


def _deg_body(dst_hbm, deg_out, dst_v, ones_v, deg_acc, s_sem):
    c = lax.axis_index("c")
    s = lax.axis_index("s")
    wid = s * NC + c

    def _fill(val):
        def _row(i, _):
            for j in range(D // 16):
                ones_v[i, pl.ds(j * 16, 16)] = jnp.full((16,), val, jnp.float32)
            return 0
        lax.fori_loop(0, EB, _row, 0)

    _fill(0.0)
    base = s * RPT
    for k in range(RPT // EB):
        pltpu.sync_copy(ones_v, deg_acc.at[pl.ds(base + k * EB, EB)])
    _fill(1.0)

    plsc.subcore_barrier()

    def _chunk(jc, _):
        off = wid * NB + jc * IC
        pltpu.sync_copy(dst_hbm.at[pl.ds(off, IC)], dst_v)

        hs = [pltpu.async_copy(ones_v, deg_acc.at[dst_v.at[j]], s_sem, add=True)
              for j in range(IC)]
        for h in hs:
            h.wait()
        return 0
    lax.fori_loop(0, NB // IC, _chunk, 0)

    plsc.subcore_barrier()

    for k in range(RPT // EB):
        r0 = base + k * EB
        pltpu.sync_copy(deg_acc.at[pl.ds(r0, EB)], deg_out.at[pl.ds(c * NP + r0, EB)])


_deg = pl.kernel(
    _deg_body,
    out_type=jax.ShapeDtypeStruct((NC * NP, D), jnp.float32),
    mesh=_MESH,
    scratch_types=[
        pltpu.VMEM((IC, EB), jnp.int32),
        pltpu.VMEM((EB, D), jnp.float32),
        pltpu.VMEM_SHARED((NP, D), jnp.float32),
        pltpu.SemaphoreType.DMA,
    ],
)

BR = 2000


def _combine_body(x_ref, s_ref, d_ref, wn_ref, ws_ref, b_ref, o_ref):
    deg = d_ref[0, :, :1] + d_ref[1, :, :1]
    invd = jnp.where(deg > 0, 1.0 / jnp.maximum(deg, 1.0), 0.0)
    mean = (s_ref[0] + s_ref[1]) * invd
    hs = lax.dot_general(x_ref[...], ws_ref[...], (((1,), (1,)), ((), ())),
                         precision=lax.Precision.HIGHEST,
                         preferred_element_type=jnp.float32)
    hn = lax.dot_general(mean, wn_ref[...], (((1,), (1,)), ((), ())),
                         precision=lax.Precision.HIGHEST,
                         preferred_element_type=jnp.float32)
    o_ref[...] = hs + hn + b_ref[...]


def _combine(h, s_parts, deg_parts, W_neigh, W_self, b):
    return pl.pallas_call(
        _combine_body,
        grid=(N // BR,),
        in_specs=[
            pl.BlockSpec((BR, D), lambda i: (i, 0)),
            pl.BlockSpec((NC, BR, D), lambda i: (0, i, 0)),
            pl.BlockSpec((NC, BR, D), lambda i: (0, i, 0)),
            pl.BlockSpec((D, D), lambda i: (0, 0)),
            pl.BlockSpec((D, D), lambda i: (0, 0)),
            pl.BlockSpec((1, D), lambda i: (0, 0)),
        ],
        out_specs=pl.BlockSpec((BR, D), lambda i: (i, 0)),
        out_shape=jax.ShapeDtypeStruct((N, D), jnp.float32),
    )(h, s_parts.reshape(NC, NP, D), deg_parts.reshape(NC, NP, D),
      W_neigh, W_self, b.reshape(1, D))


def kernel(x, edge_index, W_neigh1, W_self1, b_self1, W_neigh2, W_self2, b_self2):
    src = edge_index[0]
    dst = edge_index[1]
    pad_ids = lax.iota(jnp.int32, EP - E)
    srcp = jnp.concatenate([src, pad_ids % N]).reshape(EP // EB, EB)
    dstp = jnp.concatenate([dst, N + pad_ids % (NP - N)]).reshape(EP // EB, EB)

    d = _deg(dstp)
    s1 = _sum(x, srcp, dstp)
    h1 = _combine(x, s1, d, W_neigh1, W_self1, b_self1)
    s2 = _sum(h1, srcp, dstp)
    h2 = _combine(h1, s2, d, W_neigh2, W_self2, b_self2)
    return h2


# speedup vs baseline: 9.3080x; 1.1446x over previous
import jax
import jax.numpy as jnp
from jax import lax
from jax.experimental import pallas as pl
from jax.experimental.pallas import tpu as pltpu
from jax.experimental.pallas import tpu_sc as plsc

N = 10000
D = 128
E = 320000
NC, NS = 2, 16
NW = NC * NS
NP = 10240
EB = 64
NB = (-(-E // (NW * EB)) + 7) // 8 * 8
EP = NW * NB * EB
RPT = NP // NS
IC = 16

_MESH = plsc.VectorSubcoreMesh(core_axis_name="c", subcore_axis_name="s",
                               num_cores=NC, num_subcores=NS)


def _sum_body(h_hbm, src_hbm, dst_hbm, s_out,
              src_v, dst_v, rows_v, s_acc, g_sem0, g_sem1, s_sem0, s_sem1):
    c = lax.axis_index("c")
    s = lax.axis_index("s")
    wid = s * NC + c
    gsems = [g_sem0, g_sem1]
    ssems = [s_sem0, s_sem1]

    def _zrow(i, _):
        for j in range(D // 16):
            rows_v[0, i, pl.ds(j * 16, 16)] = jnp.zeros((16,), jnp.float32)
        return 0
    lax.fori_loop(0, EB, _zrow, 0)

    base = s * RPT
    for k in range(RPT // EB):
        pltpu.sync_copy(rows_v.at[0], s_acc.at[pl.ds(base + k * EB, EB)])

    plsc.subcore_barrier()

    def _chunk(jc, _):
        off = wid * NB + jc * IC
        pltpu.sync_copy(src_hbm.at[pl.ds(off, IC)], src_v)
        pltpu.sync_copy(dst_hbm.at[pl.ds(off, IC)], dst_v)

        g = {0: pltpu.async_copy(h_hbm.at[src_v.at[0]], rows_v.at[0], gsems[0])}
        sc = {}
        for j in range(IC):
            b = j % 2
            if j >= 1:
                sc[j - 1].wait()
            if j + 1 < IC:
                g[j + 1] = pltpu.async_copy(
                    h_hbm.at[src_v.at[j + 1]], rows_v.at[(j + 1) % 2],
                    gsems[(j + 1) % 2])
            g[j].wait()
            sc[j] = pltpu.async_copy(
                rows_v.at[b], s_acc.at[dst_v.at[j]], ssems[b], add=True)
        sc[IC - 1].wait()
        return 0
    lax.fori_loop(0, NB // IC, _chunk, 0)

    plsc.subcore_barrier()

    for k in range(RPT // EB):
        r0 = base + k * EB
        pltpu.sync_copy(s_acc.at[pl.ds(r0, EB)], s_out.at[pl.ds(c * NP + r0, EB)])


_sum = pl.kernel(
    _sum_body,
    out_type=jax.ShapeDtypeStruct((NC * NP, D), jnp.float32),
    mesh=_MESH,
    scratch_types=[
        pltpu.VMEM((IC, EB), jnp.int32),
        pltpu.VMEM((IC, EB), jnp.int32),
        pltpu.VMEM((2, EB, D), jnp.float32),
        pltpu.VMEM_SHARED((NP, D), jnp.float32),
        pltpu.SemaphoreType.DMA,
        pltpu.SemaphoreType.DMA,
        pltpu.SemaphoreType.DMA,
        pltpu.SemaphoreType.DMA,
    ],
)


NR = NP // 128
RR = RPT // 128


def _deg_body(dst_hbm, deg_out, dst_all, deg_v, blk_v, res_v, buf_v, stage):
    c = lax.axis_index("c")
    s = lax.axis_index("s")
    wid = s * NC + c

    def _z(t, _):
        for j in range(128 // 16):
            deg_v[t, pl.ds(j * 16, 16)] = jnp.zeros((16,), jnp.float32)
        return 0
    lax.fori_loop(0, NR, _z, 0)

    pltpu.sync_copy(dst_hbm.at[pl.ds(wid * NB, NB)], dst_all)
    ones16 = jnp.ones((16,), jnp.float32)

    def _sc(r, _):
        for k in range(EB // 16):
            idx = dst_all[r, pl.ds(k * 16, 16)]
            plsc.addupdate_scatter(
                deg_v, [lax.shift_right_logical(idx, 7),
                        lax.bitwise_and(idx, 127)], ones16)
        return 0
    lax.fori_loop(0, NB, _sc, 0)

    pltpu.sync_copy(deg_v, stage.at[s])
    plsc.subcore_barrier()

    for q in range(NS):
        pltpu.sync_copy(stage.at[q, pl.ds(s * RR, RR)], blk_v)
        for rr in range(RR):
            for j in range(128 // 16):
                a = blk_v[rr, pl.ds(j * 16, 16)]
                if q == 0:
                    res_v[rr, pl.ds(j * 16, 16)] = a
                else:
                    prev = res_v[rr, pl.ds(j * 16, 16)]
                    res_v[rr, pl.ds(j * 16, 16)] = prev + a

    base = s * RPT

    def _wgrp(g, _):
        rr = g // 2
        hh = g % 2
        for q in range(4):
            a = res_v[rr, pl.ds(hh * 64 + q * 16, 16)]
            for l in range(16):
                vec = a[l] * jnp.ones((16,), jnp.float32)
                for j in range(D // 16):
                    buf_v[q * 16 + l, pl.ds(j * 16, 16)] = vec
        pltpu.sync_copy(buf_v, deg_out.at[pl.ds(c * NP + base + g * 64, 64)])
        return 0
    lax.fori_loop(0, RPT // 64, _wgrp, 0)


_deg = pl.kernel(
    _deg_body,
    out_type=jax.ShapeDtypeStruct((NC * NP, D), jnp.float32),
    mesh=_MESH,
    compiler_params=pltpu.CompilerParams(needs_layout_passes=False),
    scratch_types=[
        pltpu.VMEM((NB, EB), jnp.int32),
        pltpu.VMEM((NR, 128), jnp.float32),
        pltpu.VMEM((RR, 128), jnp.float32),
        pltpu.VMEM((RR, 128), jnp.float32),
        pltpu.VMEM((64, D), jnp.float32),
        pltpu.VMEM_SHARED((NS, NR, 128), jnp.float32),
    ],
)

BR = 2000


def _combine_body(x_ref, s_ref, d_ref, wn_ref, ws_ref, b_ref, o_ref):
    deg = d_ref[0, :, :1] + d_ref[1, :, :1]
    invd = jnp.where(deg > 0, 1.0 / jnp.maximum(deg, 1.0), 0.0)
    mean = (s_ref[0] + s_ref[1]) * invd
    hs = lax.dot_general(x_ref[...], ws_ref[...], (((1,), (1,)), ((), ())),
                         precision=lax.Precision.HIGHEST,
                         preferred_element_type=jnp.float32)
    hn = lax.dot_general(mean, wn_ref[...], (((1,), (1,)), ((), ())),
                         precision=lax.Precision.HIGHEST,
                         preferred_element_type=jnp.float32)
    o_ref[...] = hs + hn + b_ref[...]


def _combine(h, s_parts, deg_parts, W_neigh, W_self, b):
    return pl.pallas_call(
        _combine_body,
        grid=(N // BR,),
        in_specs=[
            pl.BlockSpec((BR, D), lambda i: (i, 0)),
            pl.BlockSpec((NC, BR, D), lambda i: (0, i, 0)),
            pl.BlockSpec((NC, BR, D), lambda i: (0, i, 0)),
            pl.BlockSpec((D, D), lambda i: (0, 0)),
            pl.BlockSpec((D, D), lambda i: (0, 0)),
            pl.BlockSpec((1, D), lambda i: (0, 0)),
        ],
        out_specs=pl.BlockSpec((BR, D), lambda i: (i, 0)),
        out_shape=jax.ShapeDtypeStruct((N, D), jnp.float32),
    )(h, s_parts.reshape(NC, NP, D), deg_parts.reshape(NC, NP, D),
      W_neigh, W_self, b.reshape(1, D))


def kernel(x, edge_index, W_neigh1, W_self1, b_self1, W_neigh2, W_self2, b_self2):
    src = edge_index[0]
    dst = edge_index[1]
    pad_ids = lax.iota(jnp.int32, EP - E)
    srcp = jnp.concatenate([src, pad_ids % N]).reshape(EP // EB, EB)
    dstp = jnp.concatenate([dst, N + pad_ids % (NP - N)]).reshape(EP // EB, EB)

    d = _deg(dstp)
    s1 = _sum(x, srcp, dstp)
    h1 = _combine(x, s1, d, W_neigh1, W_self1, b_self1)
    s2 = _sum(h1, srcp, dstp)
    h2 = _combine(h1, s2, d, W_neigh2, W_self2, b_self2)
    return h2


# speedup vs baseline: 9.6635x; 1.0382x over previous
import jax
import jax.numpy as jnp
from jax import lax
from jax.experimental import pallas as pl
from jax.experimental.pallas import tpu as pltpu
from jax.experimental.pallas import tpu_sc as plsc

N = 10000
D = 128
E = 320000
NC, NS = 2, 16
NW = NC * NS
NP = 10240
EB = 64
NB = (-(-E // (NW * EB)) + 7) // 8 * 8
EP = NW * NB * EB
RPT = NP // NS
IC = 32

_MESH = plsc.VectorSubcoreMesh(core_axis_name="c", subcore_axis_name="s",
                               num_cores=NC, num_subcores=NS)


def _sum_body(h_hbm, src_hbm, dst_hbm, s_out,
              src_v, dst_v, rows_v, s_acc, g_sem0, g_sem1, s_sem0, s_sem1):
    c = lax.axis_index("c")
    s = lax.axis_index("s")
    wid = s * NC + c
    gsems = [g_sem0, g_sem1]
    ssems = [s_sem0, s_sem1]

    def _zrow(i, _):
        for j in range(D // 16):
            rows_v[0, i, pl.ds(j * 16, 16)] = jnp.zeros((16,), jnp.float32)
        return 0
    lax.fori_loop(0, EB, _zrow, 0)

    base = s * RPT
    for k in range(RPT // EB):
        pltpu.sync_copy(rows_v.at[0], s_acc.at[pl.ds(base + k * EB, EB)])

    plsc.subcore_barrier()

    def _chunk(jc, _):
        off = wid * NB + jc * IC
        pltpu.sync_copy(src_hbm.at[pl.ds(off, IC)], src_v)
        pltpu.sync_copy(dst_hbm.at[pl.ds(off, IC)], dst_v)

        g = {0: pltpu.async_copy(h_hbm.at[src_v.at[0]], rows_v.at[0], gsems[0])}
        sc = {}
        for j in range(IC):
            b = j % 2
            if j >= 1:
                sc[j - 1].wait()
            if j + 1 < IC:
                g[j + 1] = pltpu.async_copy(
                    h_hbm.at[src_v.at[j + 1]], rows_v.at[(j + 1) % 2],
                    gsems[(j + 1) % 2])
            g[j].wait()
            sc[j] = pltpu.async_copy(
                rows_v.at[b], s_acc.at[dst_v.at[j]], ssems[b], add=True)
        sc[IC - 1].wait()
        return 0
    lax.fori_loop(0, NB // IC, _chunk, 0)

    plsc.subcore_barrier()

    for k in range(RPT // EB):
        r0 = base + k * EB
        pltpu.sync_copy(s_acc.at[pl.ds(r0, EB)], s_out.at[pl.ds(c * NP + r0, EB)])


_sum = pl.kernel(
    _sum_body,
    out_type=jax.ShapeDtypeStruct((NC * NP, D), jnp.float32),
    mesh=_MESH,
    scratch_types=[
        pltpu.VMEM((IC, EB), jnp.int32),
        pltpu.VMEM((IC, EB), jnp.int32),
        pltpu.VMEM((2, EB, D), jnp.float32),
        pltpu.VMEM_SHARED((NP, D), jnp.float32),
        pltpu.SemaphoreType.DMA,
        pltpu.SemaphoreType.DMA,
        pltpu.SemaphoreType.DMA,
        pltpu.SemaphoreType.DMA,
    ],
)


NR = NP // 128
RR = RPT // 128


def _deg_body(dst_hbm, deg_out, dst_all, deg_v, blk_v, res_v, buf_v, stage):
    c = lax.axis_index("c")
    s = lax.axis_index("s")
    wid = s * NC + c

    def _z(t, _):
        for j in range(128 // 16):
            deg_v[t, pl.ds(j * 16, 16)] = jnp.zeros((16,), jnp.float32)
        return 0
    lax.fori_loop(0, NR, _z, 0)

    pltpu.sync_copy(dst_hbm.at[pl.ds(wid * NB, NB)], dst_all)
    ones16 = jnp.ones((16,), jnp.float32)

    def _sc(r, _):
        for k in range(EB // 16):
            idx = dst_all[r, pl.ds(k * 16, 16)]
            plsc.addupdate_scatter(
                deg_v, [lax.shift_right_logical(idx, 7),
                        lax.bitwise_and(idx, 127)], ones16)
        return 0
    lax.fori_loop(0, NB, _sc, 0)

    pltpu.sync_copy(deg_v, stage.at[s])
    plsc.subcore_barrier()

    for q in range(NS):
        pltpu.sync_copy(stage.at[q, pl.ds(s * RR, RR)], blk_v)
        for rr in range(RR):
            for j in range(128 // 16):
                a = blk_v[rr, pl.ds(j * 16, 16)]
                if q == 0:
                    res_v[rr, pl.ds(j * 16, 16)] = a
                else:
                    prev = res_v[rr, pl.ds(j * 16, 16)]
                    res_v[rr, pl.ds(j * 16, 16)] = prev + a

    base = s * RPT

    def _wgrp(g, _):
        rr = g // 2
        hh = g % 2
        for q in range(4):
            a = res_v[rr, pl.ds(hh * 64 + q * 16, 16)]
            for l in range(16):
                vec = a[l] * jnp.ones((16,), jnp.float32)
                for j in range(D // 16):
                    buf_v[q * 16 + l, pl.ds(j * 16, 16)] = vec
        pltpu.sync_copy(buf_v, deg_out.at[pl.ds(c * NP + base + g * 64, 64)])
        return 0
    lax.fori_loop(0, RPT // 64, _wgrp, 0)


_deg = pl.kernel(
    _deg_body,
    out_type=jax.ShapeDtypeStruct((NC * NP, D), jnp.float32),
    mesh=_MESH,
    compiler_params=pltpu.CompilerParams(needs_layout_passes=False),
    scratch_types=[
        pltpu.VMEM((NB, EB), jnp.int32),
        pltpu.VMEM((NR, 128), jnp.float32),
        pltpu.VMEM((RR, 128), jnp.float32),
        pltpu.VMEM((RR, 128), jnp.float32),
        pltpu.VMEM((64, D), jnp.float32),
        pltpu.VMEM_SHARED((NS, NR, 128), jnp.float32),
    ],
)

BR = 2000


def _combine_body(x_ref, s_ref, d_ref, wn_ref, ws_ref, b_ref, o_ref):
    deg = d_ref[0, :, :1] + d_ref[1, :, :1]
    invd = jnp.where(deg > 0, 1.0 / jnp.maximum(deg, 1.0), 0.0)
    mean = (s_ref[0] + s_ref[1]) * invd
    hs = lax.dot_general(x_ref[...], ws_ref[...], (((1,), (1,)), ((), ())),
                         precision=lax.Precision.HIGHEST,
                         preferred_element_type=jnp.float32)
    hn = lax.dot_general(mean, wn_ref[...], (((1,), (1,)), ((), ())),
                         precision=lax.Precision.HIGHEST,
                         preferred_element_type=jnp.float32)
    o_ref[...] = hs + hn + b_ref[...]


def _combine(h, s_parts, deg_parts, W_neigh, W_self, b):
    return pl.pallas_call(
        _combine_body,
        grid=(N // BR,),
        in_specs=[
            pl.BlockSpec((BR, D), lambda i: (i, 0)),
            pl.BlockSpec((NC, BR, D), lambda i: (0, i, 0)),
            pl.BlockSpec((NC, BR, D), lambda i: (0, i, 0)),
            pl.BlockSpec((D, D), lambda i: (0, 0)),
            pl.BlockSpec((D, D), lambda i: (0, 0)),
            pl.BlockSpec((1, D), lambda i: (0, 0)),
        ],
        out_specs=pl.BlockSpec((BR, D), lambda i: (i, 0)),
        out_shape=jax.ShapeDtypeStruct((N, D), jnp.float32),
    )(h, s_parts.reshape(NC, NP, D), deg_parts.reshape(NC, NP, D),
      W_neigh, W_self, b.reshape(1, D))


def kernel(x, edge_index, W_neigh1, W_self1, b_self1, W_neigh2, W_self2, b_self2):
    src = edge_index[0]
    dst = edge_index[1]
    pad_ids = lax.iota(jnp.int32, EP - E)
    srcp = jnp.concatenate([src, pad_ids % N]).reshape(EP // EB, EB)
    dstp = jnp.concatenate([dst, N + pad_ids % (NP - N)]).reshape(EP // EB, EB)

    d = _deg(dstp)
    s1 = _sum(x, srcp, dstp)
    h1 = _combine(x, s1, d, W_neigh1, W_self1, b_self1)
    s2 = _sum(h1, srcp, dstp)
    h2 = _combine(h1, s2, d, W_neigh2, W_self2, b_self2)
    return h2


# speedup vs baseline: 9.6972x; 1.0035x over previous
import jax
import jax.numpy as jnp
from jax import lax
from jax.experimental import pallas as pl
from jax.experimental.pallas import tpu as pltpu
from jax.experimental.pallas import tpu_sc as plsc

N = 10000
D = 128
E = 320000
NC, NS = 2, 16
NW = NC * NS
NP = 10240
EB = 64
NB = (-(-E // (NW * EB)) + 7) // 8 * 8
EP = NW * NB * EB
RPT = NP // NS
IC = 32

_MESH = plsc.VectorSubcoreMesh(core_axis_name="c", subcore_axis_name="s",
                               num_cores=NC, num_subcores=NS)


def _sum_body(h_hbm, src_hbm, dst_hbm, s_out,
              src_v, dst_v, rows_v, s_acc, g_sem0, g_sem1, s_sem0, s_sem1):
    c = lax.axis_index("c")
    s = lax.axis_index("s")
    wid = s * NC + c
    gsems = [g_sem0, g_sem1]
    ssems = [s_sem0, s_sem1]

    def _zrow(i, _):
        for j in range(D // 16):
            rows_v[0, i, pl.ds(j * 16, 16)] = jnp.zeros((16,), jnp.float32)
        return 0
    lax.fori_loop(0, EB, _zrow, 0)

    base = s * RPT
    zs = [pltpu.async_copy(rows_v.at[0], s_acc.at[pl.ds(base + k * EB, EB)],
                           gsems[k % 2])
          for k in range(RPT // EB)]
    for z in zs:
        z.wait()

    plsc.subcore_barrier()

    def _chunk(jc, _):
        off = wid * NB + jc * IC
        pltpu.sync_copy(src_hbm.at[pl.ds(off, IC)], src_v)
        pltpu.sync_copy(dst_hbm.at[pl.ds(off, IC)], dst_v)

        g = {0: pltpu.async_copy(h_hbm.at[src_v.at[0]], rows_v.at[0], gsems[0])}
        sc = {}
        for j in range(IC):
            b = j % 2
            if j >= 1:
                sc[j - 1].wait()
            if j + 1 < IC:
                g[j + 1] = pltpu.async_copy(
                    h_hbm.at[src_v.at[j + 1]], rows_v.at[(j + 1) % 2],
                    gsems[(j + 1) % 2])
            g[j].wait()
            sc[j] = pltpu.async_copy(
                rows_v.at[b], s_acc.at[dst_v.at[j]], ssems[b], add=True)
        sc[IC - 1].wait()
        return 0
    lax.fori_loop(0, NB // IC, _chunk, 0)

    plsc.subcore_barrier()

    for k in range(RPT // EB):
        r0 = base + k * EB
        pltpu.sync_copy(s_acc.at[pl.ds(r0, EB)], s_out.at[pl.ds(c * NP + r0, EB)])


_sum = pl.kernel(
    _sum_body,
    out_type=jax.ShapeDtypeStruct((NC * NP, D), jnp.float32),
    mesh=_MESH,
    scratch_types=[
        pltpu.VMEM((IC, EB), jnp.int32),
        pltpu.VMEM((IC, EB), jnp.int32),
        pltpu.VMEM((2, EB, D), jnp.float32),
        pltpu.VMEM_SHARED((NP, D), jnp.float32),
        pltpu.SemaphoreType.DMA,
        pltpu.SemaphoreType.DMA,
        pltpu.SemaphoreType.DMA,
        pltpu.SemaphoreType.DMA,
    ],
)


NR = NP // 128
RR = RPT // 128


def _deg_body(dst_hbm, deg_out, dst_all, deg_v, blk_v, res_v, buf_v, stage):
    c = lax.axis_index("c")
    s = lax.axis_index("s")
    wid = s * NC + c

    def _z(t, _):
        for j in range(128 // 16):
            deg_v[t, pl.ds(j * 16, 16)] = jnp.zeros((16,), jnp.float32)
        return 0
    lax.fori_loop(0, NR, _z, 0)

    pltpu.sync_copy(dst_hbm.at[pl.ds(wid * NB, NB)], dst_all)
    ones16 = jnp.ones((16,), jnp.float32)

    def _sc(r, _):
        for k in range(EB // 16):
            idx = dst_all[r, pl.ds(k * 16, 16)]
            plsc.addupdate_scatter(
                deg_v, [lax.shift_right_logical(idx, 7),
                        lax.bitwise_and(idx, 127)], ones16)
        return 0
    lax.fori_loop(0, NB, _sc, 0)

    pltpu.sync_copy(deg_v, stage.at[s])
    plsc.subcore_barrier()

    for q in range(NS):
        pltpu.sync_copy(stage.at[q, pl.ds(s * RR, RR)], blk_v)
        for rr in range(RR):
            for j in range(128 // 16):
                a = blk_v[rr, pl.ds(j * 16, 16)]
                if q == 0:
                    res_v[rr, pl.ds(j * 16, 16)] = a
                else:
                    prev = res_v[rr, pl.ds(j * 16, 16)]
                    res_v[rr, pl.ds(j * 16, 16)] = prev + a

    base = s * RPT

    def _wgrp(g, _):
        rr = g // 2
        hh = g % 2
        for q in range(4):
            a = res_v[rr, pl.ds(hh * 64 + q * 16, 16)]
            for l in range(16):
                vec = a[l] * jnp.ones((16,), jnp.float32)
                for j in range(D // 16):
                    buf_v[q * 16 + l, pl.ds(j * 16, 16)] = vec
        pltpu.sync_copy(buf_v, deg_out.at[pl.ds(c * NP + base + g * 64, 64)])
        return 0
    lax.fori_loop(0, RPT // 64, _wgrp, 0)


_deg = pl.kernel(
    _deg_body,
    out_type=jax.ShapeDtypeStruct((NC * NP, D), jnp.float32),
    mesh=_MESH,
    compiler_params=pltpu.CompilerParams(needs_layout_passes=False),
    scratch_types=[
        pltpu.VMEM((NB, EB), jnp.int32),
        pltpu.VMEM((NR, 128), jnp.float32),
        pltpu.VMEM((RR, 128), jnp.float32),
        pltpu.VMEM((RR, 128), jnp.float32),
        pltpu.VMEM((64, D), jnp.float32),
        pltpu.VMEM_SHARED((NS, NR, 128), jnp.float32),
    ],
)

BR = 2000


def _combine_body(x_ref, s_ref, d_ref, wn_ref, ws_ref, b_ref, o_ref):
    deg = d_ref[0, :, :1] + d_ref[1, :, :1]
    invd = jnp.where(deg > 0, 1.0 / jnp.maximum(deg, 1.0), 0.0)
    mean = (s_ref[0] + s_ref[1]) * invd
    hs = lax.dot_general(x_ref[...], ws_ref[...], (((1,), (1,)), ((), ())),
                         precision=lax.Precision.HIGHEST,
                         preferred_element_type=jnp.float32)
    hn = lax.dot_general(mean, wn_ref[...], (((1,), (1,)), ((), ())),
                         precision=lax.Precision.HIGHEST,
                         preferred_element_type=jnp.float32)
    o_ref[...] = hs + hn + b_ref[...]


def _combine(h, s_parts, deg_parts, W_neigh, W_self, b):
    return pl.pallas_call(
        _combine_body,
        grid=(N // BR,),
        in_specs=[
            pl.BlockSpec((BR, D), lambda i: (i, 0)),
            pl.BlockSpec((NC, BR, D), lambda i: (0, i, 0)),
            pl.BlockSpec((NC, BR, D), lambda i: (0, i, 0)),
            pl.BlockSpec((D, D), lambda i: (0, 0)),
            pl.BlockSpec((D, D), lambda i: (0, 0)),
            pl.BlockSpec((1, D), lambda i: (0, 0)),
        ],
        out_specs=pl.BlockSpec((BR, D), lambda i: (i, 0)),
        out_shape=jax.ShapeDtypeStruct((N, D), jnp.float32),
    )(h, s_parts.reshape(NC, NP, D), deg_parts.reshape(NC, NP, D),
      W_neigh, W_self, b.reshape(1, D))


def kernel(x, edge_index, W_neigh1, W_self1, b_self1, W_neigh2, W_self2, b_self2):
    src = edge_index[0]
    dst = edge_index[1]
    pad_ids = lax.iota(jnp.int32, EP - E)
    srcp = jnp.concatenate([src, pad_ids % N]).reshape(EP // EB, EB)
    dstp = jnp.concatenate([dst, N + pad_ids % (NP - N)]).reshape(EP // EB, EB)

    d = _deg(dstp)
    s1 = _sum(x, srcp, dstp)
    h1 = _combine(x, s1, d, W_neigh1, W_self1, b_self1)
    s2 = _sum(h1, srcp, dstp)
    h2 = _combine(h1, s2, d, W_neigh2, W_self2, b_self2)
    return h2


# speedup vs baseline: 9.7494x; 1.0054x over previous
import jax
import jax.numpy as jnp
from jax import lax
from jax.experimental import pallas as pl
from jax.experimental.pallas import tpu as pltpu
from jax.experimental.pallas import tpu_sc as plsc

N = 10000
D = 128
E = 320000
NC, NS = 2, 16
NW = NC * NS
NP = 10240
EB = 64
NB = (-(-E // (NW * EB)) + 7) // 8 * 8
EP = NW * NB * EB
RPT = NP // NS
IC = 32

_MESH = plsc.VectorSubcoreMesh(core_axis_name="c", subcore_axis_name="s",
                               num_cores=NC, num_subcores=NS)


def _sum_body(h_hbm, src_hbm, dst_hbm, s_out,
              src_v, dst_v, rows_v, s_acc, g_sem0, g_sem1, s_sem0, s_sem1):
    c = lax.axis_index("c")
    s = lax.axis_index("s")
    wid = s * NC + c
    gsems = [g_sem0, g_sem1]
    ssems = [s_sem0, s_sem1]

    def _zrow(i, _):
        for j in range(D // 16):
            rows_v[0, i, pl.ds(j * 16, 16)] = jnp.zeros((16,), jnp.float32)
        return 0
    lax.fori_loop(0, EB, _zrow, 0)

    base = s * RPT
    zs = [pltpu.async_copy(rows_v.at[0], s_acc.at[pl.ds(base + k * EB, EB)],
                           gsems[k % 2])
          for k in range(RPT // EB)]
    for z in zs:
        z.wait()

    plsc.subcore_barrier()

    def _chunk(jc, _):
        off = wid * NB + jc * IC
        pltpu.sync_copy(src_hbm.at[pl.ds(off, IC)], src_v)
        pltpu.sync_copy(dst_hbm.at[pl.ds(off, IC)], dst_v)

        g = {0: pltpu.async_copy(h_hbm.at[src_v.at[0]], rows_v.at[0], gsems[0])}
        sc = {}
        for j in range(IC):
            b = j % 2
            if j >= 1:
                sc[j - 1].wait()
            if j + 1 < IC:
                g[j + 1] = pltpu.async_copy(
                    h_hbm.at[src_v.at[j + 1]], rows_v.at[(j + 1) % 2],
                    gsems[(j + 1) % 2])
            g[j].wait()
            sc[j] = pltpu.async_copy(
                rows_v.at[b], s_acc.at[dst_v.at[j]], ssems[b], add=True)
        sc[IC - 1].wait()
        return 0
    lax.fori_loop(0, NB // IC, _chunk, 0)

    plsc.subcore_barrier()

    ws = [pltpu.async_copy(s_acc.at[pl.ds(base + k * EB, EB)],
                           s_out.at[pl.ds(c * NP + base + k * EB, EB)],
                           gsems[k % 2])
          for k in range(RPT // EB)]
    for w in ws:
        w.wait()


_sum = pl.kernel(
    _sum_body,
    out_type=jax.ShapeDtypeStruct((NC * NP, D), jnp.float32),
    mesh=_MESH,
    scratch_types=[
        pltpu.VMEM((IC, EB), jnp.int32),
        pltpu.VMEM((IC, EB), jnp.int32),
        pltpu.VMEM((2, EB, D), jnp.float32),
        pltpu.VMEM_SHARED((NP, D), jnp.float32),
        pltpu.SemaphoreType.DMA,
        pltpu.SemaphoreType.DMA,
        pltpu.SemaphoreType.DMA,
        pltpu.SemaphoreType.DMA,
    ],
)


NR = NP // 128
RR = RPT // 128


def _deg_body(dst_hbm, deg_out, dst_all, deg_v, blk_v, res_v, buf_v, stage):
    c = lax.axis_index("c")
    s = lax.axis_index("s")
    wid = s * NC + c

    def _z(t, _):
        for j in range(128 // 16):
            deg_v[t, pl.ds(j * 16, 16)] = jnp.zeros((16,), jnp.float32)
        return 0
    lax.fori_loop(0, NR, _z, 0)

    pltpu.sync_copy(dst_hbm.at[pl.ds(wid * NB, NB)], dst_all)
    ones16 = jnp.ones((16,), jnp.float32)

    def _sc(r, _):
        for k in range(EB // 16):
            idx = dst_all[r, pl.ds(k * 16, 16)]
            plsc.addupdate_scatter(
                deg_v, [lax.shift_right_logical(idx, 7),
                        lax.bitwise_and(idx, 127)], ones16)
        return 0
    lax.fori_loop(0, NB, _sc, 0)

    pltpu.sync_copy(deg_v, stage.at[s])
    plsc.subcore_barrier()

    for q in range(NS):
        pltpu.sync_copy(stage.at[q, pl.ds(s * RR, RR)], blk_v)
        for rr in range(RR):
            for j in range(128 // 16):
                a = blk_v[rr, pl.ds(j * 16, 16)]
                if q == 0:
                    res_v[rr, pl.ds(j * 16, 16)] = a
                else:
                    prev = res_v[rr, pl.ds(j * 16, 16)]
                    res_v[rr, pl.ds(j * 16, 16)] = prev + a

    base = s * RPT

    def _wgrp(g, _):
        rr = g // 2
        hh = g % 2
        for q in range(4):
            a = res_v[rr, pl.ds(hh * 64 + q * 16, 16)]
            for l in range(16):
                vec = a[l] * jnp.ones((16,), jnp.float32)
                for j in range(D // 16):
                    buf_v[q * 16 + l, pl.ds(j * 16, 16)] = vec
        pltpu.sync_copy(buf_v, deg_out.at[pl.ds(c * NP + base + g * 64, 64)])
        return 0
    lax.fori_loop(0, RPT // 64, _wgrp, 0)


_deg = pl.kernel(
    _deg_body,
    out_type=jax.ShapeDtypeStruct((NC * NP, D), jnp.float32),
    mesh=_MESH,
    compiler_params=pltpu.CompilerParams(needs_layout_passes=False),
    scratch_types=[
        pltpu.VMEM((NB, EB), jnp.int32),
        pltpu.VMEM((NR, 128), jnp.float32),
        pltpu.VMEM((RR, 128), jnp.float32),
        pltpu.VMEM((RR, 128), jnp.float32),
        pltpu.VMEM((64, D), jnp.float32),
        pltpu.VMEM_SHARED((NS, NR, 128), jnp.float32),
    ],
)

BR = 2000


def _combine_body(x_ref, s_ref, d_ref, wn_ref, ws_ref, b_ref, o_ref):
    deg = d_ref[0, :, :1] + d_ref[1, :, :1]
    invd = jnp.where(deg > 0, 1.0 / jnp.maximum(deg, 1.0), 0.0)
    mean = (s_ref[0] + s_ref[1]) * invd
    hs = lax.dot_general(x_ref[...], ws_ref[...], (((1,), (1,)), ((), ())),
                         precision=lax.Precision.HIGHEST,
                         preferred_element_type=jnp.float32)
    hn = lax.dot_general(mean, wn_ref[...], (((1,), (1,)), ((), ())),
                         precision=lax.Precision.HIGHEST,
                         preferred_element_type=jnp.float32)
    o_ref[...] = hs + hn + b_ref[...]


def _combine(h, s_parts, deg_parts, W_neigh, W_self, b):
    return pl.pallas_call(
        _combine_body,
        grid=(N // BR,),
        in_specs=[
            pl.BlockSpec((BR, D), lambda i: (i, 0)),
            pl.BlockSpec((NC, BR, D), lambda i: (0, i, 0)),
            pl.BlockSpec((NC, BR, D), lambda i: (0, i, 0)),
            pl.BlockSpec((D, D), lambda i: (0, 0)),
            pl.BlockSpec((D, D), lambda i: (0, 0)),
            pl.BlockSpec((1, D), lambda i: (0, 0)),
        ],
        out_specs=pl.BlockSpec((BR, D), lambda i: (i, 0)),
        out_shape=jax.ShapeDtypeStruct((N, D), jnp.float32),
    )(h, s_parts.reshape(NC, NP, D), deg_parts.reshape(NC, NP, D),
      W_neigh, W_self, b.reshape(1, D))


def kernel(x, edge_index, W_neigh1, W_self1, b_self1, W_neigh2, W_self2, b_self2):
    src = edge_index[0]
    dst = edge_index[1]
    pad_ids = lax.iota(jnp.int32, EP - E)
    srcp = jnp.concatenate([src, pad_ids % N]).reshape(EP // EB, EB)
    dstp = jnp.concatenate([dst, N + pad_ids % (NP - N)]).reshape(EP // EB, EB)

    d = _deg(dstp)
    s1 = _sum(x, srcp, dstp)
    h1 = _combine(x, s1, d, W_neigh1, W_self1, b_self1)
    s2 = _sum(h1, srcp, dstp)
    h2 = _combine(h1, s2, d, W_neigh2, W_self2, b_self2)
    return h2
